# Initial kernel scaffold; baseline (speedup 1.0000x reference)
import jax
import jax.numpy as jnp
from jax.experimental import pallas as pl


def kernel(idx, embedding):
    raise NotImplementedError("write your pallas kernel here")


# speedup vs baseline: 1.4816x; 1.4816x over previous
import functools

import jax
import jax.numpy as jnp
from jax import lax
from jax.experimental import pallas as pl
from jax.experimental.pallas import tpu as pltpu
from jax.experimental.pallas import tpu_sc as plsc

D = 32
G = 128
NC, NS = 2, 16
NW = NC * NS


@functools.lru_cache(maxsize=None)
def _build(B, V):
    b_per_w = B // NW
    n_groups = b_per_w // G
    C_G = 10
    C = C_G * G
    n_chunks = n_groups // C_G
    assert b_per_w * NW == B and n_groups * G == b_per_w
    assert n_chunks * C_G == n_groups

    mesh = plsc.VectorSubcoreMesh(core_axis_name="c", subcore_axis_name="s")

    @functools.partial(
        pl.kernel,
        out_type=jax.ShapeDtypeStruct((B, D), jnp.float32),
        mesh=mesh,
        scratch_types=[
            pltpu.VMEM((n_groups, G), jnp.int32),
            pltpu.VMEM((C, D), jnp.float32),
            pltpu.SemaphoreType.DMA,
        ],
        compiler_params=pltpu.CompilerParams(use_tc_tiling_on_sc=False),
    )
    def gather_kernel(idx_hbm, table_hbm, out_hbm, idx_v, rows_v, gsem):
        wid = lax.axis_index("s") * NC + lax.axis_index("c")
        base = wid * b_per_w
        pltpu.sync_copy(idx_hbm.at[pl.ds(wid * n_groups, n_groups)], idx_v)

        def chunk_body(ci, carry):
            handles = []
            for g in range(C_G):
                h = pltpu.async_copy(
                    table_hbm.at[idx_v.at[ci * C_G + g]],
                    rows_v.at[pl.ds(g * G, G)],
                    gsem,
                )
                handles.append(h)
            for h in handles:
                h.wait()
            pltpu.sync_copy(rows_v, out_hbm.at[pl.ds(base + ci * C, C)])
            return carry

        lax.fori_loop(0, n_chunks, chunk_body, 0)

    return gather_kernel


def kernel(idx, embedding):
    Bt, H = idx.shape
    B = Bt * H
    V, d = embedding.shape
    idx2d = idx.reshape(B // G, G)
    out = _build(B, V)(idx2d, embedding)
    return out.reshape(Bt, H, d)


# speedup vs baseline: 1.4988x; 1.0116x over previous
import functools

import jax
import jax.numpy as jnp
from jax import lax
from jax.experimental import pallas as pl
from jax.experimental.pallas import tpu as pltpu
from jax.experimental.pallas import tpu_sc as plsc

D = 32
NC, NS = 2, 16
NW = NC * NS
C = 3200


@functools.lru_cache(maxsize=None)
def _build(B, V):
    b_per_w = B // NW
    n_chunks = b_per_w // C
    assert b_per_w * NW == B and n_chunks * C == b_per_w

    mesh = plsc.VectorSubcoreMesh(core_axis_name="c", subcore_axis_name="s")

    @functools.partial(
        pl.kernel,
        out_type=jax.ShapeDtypeStruct((B, D), jnp.float32),
        mesh=mesh,
        scratch_types=[
            pltpu.VMEM((b_per_w,), jnp.int32),
            pltpu.VMEM((C, D), jnp.float32),
            pltpu.SemaphoreType.DMA,
        ],
        compiler_params=pltpu.CompilerParams(use_tc_tiling_on_sc=False),
    )
    def gather_kernel(idx_hbm, table_hbm, out_hbm, idx_v, rows_v, gsem):
        wid = lax.axis_index("s") * NC + lax.axis_index("c")
        base = wid * b_per_w
        pltpu.sync_copy(idx_hbm.at[pl.ds(base, b_per_w)], idx_v)

        def chunk_body(ci, carry):
            pltpu.async_copy(
                table_hbm.at[idx_v.at[pl.ds(ci * C, C)]],
                rows_v,
                gsem,
            ).wait()
            pltpu.sync_copy(rows_v, out_hbm.at[pl.ds(base + ci * C, C)])
            return carry

        lax.fori_loop(0, n_chunks, chunk_body, 0)

    return gather_kernel


def kernel(idx, embedding):
    Bt, H = idx.shape
    B = Bt * H
    V, d = embedding.shape
    out = _build(B, V)(idx.reshape(B), embedding)
    return out.reshape(Bt, H, d)
